# baseline (speedup 1.0000x reference)
import functools

import jax
import jax.numpy as jnp
from jax import lax
from jax.experimental import pallas as pl
from jax.experimental.pallas import tpu as pltpu
from jax.experimental.pallas import tpu_sc as plsc

N = 10000
E = 640000
R = 8
NB = 30
H1 = 64
H2 = 32

NC = 2
NS = 16
NW = NC * NS
EPW = E // NW
CHUNK = 128
NCHUNK = 158
PADW = NCHUNK * CHUNK - EPW
NP = 10240
RPT = NP // NS
LANES = 16
DW = 8

_mesh = plsc.VectorSubcoreMesh(
    core_axis_name="c", subcore_axis_name="s", num_cores=NC, num_subcores=NS)


BH = 8


def _w1_body(comp1_ref, basis_ref, out_ref):
    for hh in range(BH):
        out_ref[:, hh, :] = jnp.dot(comp1_ref[...], basis_ref[:, hh, :],
                                    preferred_element_type=jnp.float32)


def _build_w1(comp1, basis1_t):
    return pl.pallas_call(
        _w1_body,
        grid=(H1 // BH,),
        in_specs=[
            pl.BlockSpec((R, NB), lambda j: (0, 0)),
            pl.BlockSpec((NB, BH, N), lambda j: (0, j, 0)),
        ],
        out_specs=pl.BlockSpec((R, BH, N), lambda j: (0, j, 0)),
        out_shape=jax.ShapeDtypeStruct((R, H1, N), jnp.float32),
    )(comp1, basis1_t)


def _w2_body(comp2_ref, basis_ref, out_ref):
    out_ref[...] = jnp.dot(comp2_ref[...], basis_ref[...],
                           preferred_element_type=jnp.float32)


def _build_w2(comp2, basis2_flat):
    return pl.pallas_call(
        _w2_body,
        out_shape=jax.ShapeDtypeStruct((R, H1 * H2), jnp.float32),
    )(comp2, basis2_flat)


def _gidx_body(src_ref, rel_ref, dst_ref, g1_ref, dstp_ref):
    gpad = jnp.zeros((8, PADW), jnp.int32)
    dpad = jnp.full((8, PADW), NP - 1, jnp.int32)
    g = jnp.concatenate([rel_ref[...] * N + src_ref[...], gpad], axis=1)
    d = jnp.concatenate([dst_ref[...], dpad], axis=1)
    g1_ref[...] = g.reshape(8, NCHUNK, CHUNK)
    dstp_ref[...] = d.reshape(8, NCHUNK, CHUNK)


def _build_gidx(src2d, rel2d, dst2d):
    return pl.pallas_call(
        _gidx_body,
        grid=(NW // 8,),
        in_specs=[
            pl.BlockSpec((8, EPW), lambda i: (i, 0)),
            pl.BlockSpec((8, EPW), lambda i: (i, 0)),
            pl.BlockSpec((8, EPW), lambda i: (i, 0)),
        ],
        out_specs=[
            pl.BlockSpec((8, NCHUNK, CHUNK), lambda i: (i, 0, 0)),
            pl.BlockSpec((8, NCHUNK, CHUNK), lambda i: (i, 0, 0)),
        ],
        out_shape=[
            jax.ShapeDtypeStruct((NW, NCHUNK, CHUNK), jnp.int32),
            jax.ShapeDtypeStruct((NW, NCHUNK, CHUNK), jnp.int32),
        ],
    )(src2d, rel2d, dst2d)


def _h1_body(aggp_ref, degp_ref, root1_ref, bias1_ref, w2cat_ref, root2_ref,
             xw_ref, hroot_ref):
    dcol = (degp_ref[0] + degp_ref[1])[:, 0:1]
    invd = 1.0 / jnp.maximum(dcol, 1.0)
    a = aggp_ref[0] + aggp_ref[1]
    h1 = jnp.maximum(a * invd + root1_ref[...] + bias1_ref[...], 0.0)
    for r in range(R):
        xw_ref[r] = jnp.dot(h1, w2cat_ref[r], preferred_element_type=jnp.float32)
    hroot_ref[...] = jnp.dot(h1, root2_ref[...], preferred_element_type=jnp.float32)


def _build_h1(aggp, degp, root1, bias1_2d, w2cat, root2):
    BN = 1000
    return pl.pallas_call(
        _h1_body,
        grid=(N // BN,),
        in_specs=[
            pl.BlockSpec((NC, BN, H1), lambda i: (0, i, 0)),
            pl.BlockSpec((NC, BN, DW), lambda i: (0, i, 0)),
            pl.BlockSpec((BN, H1), lambda i: (i, 0)),
            pl.BlockSpec((1, H1), lambda i: (0, 0)),
            pl.BlockSpec((R, H1, H2), lambda i: (0, 0, 0)),
            pl.BlockSpec((H1, H2), lambda i: (0, 0)),
        ],
        out_specs=[
            pl.BlockSpec((R, BN, H2), lambda i: (0, i, 0)),
            pl.BlockSpec((BN, H2), lambda i: (i, 0)),
        ],
        out_shape=[
            jax.ShapeDtypeStruct((R, N, H2), jnp.float32),
            jax.ShapeDtypeStruct((N, H2), jnp.float32),
        ],
    )(aggp, degp, root1, bias1_2d, w2cat, root2)


def _out_body(agg2p_ref, degp_ref, hroot_ref, bias2_ref, out_ref):
    dcol = (degp_ref[0] + degp_ref[1])[:, 0:1]
    invd = 1.0 / jnp.maximum(dcol, 1.0)
    out_ref[...] = ((agg2p_ref[0] + agg2p_ref[1]) * invd
                    + hroot_ref[...] + bias2_ref[...])


def _build_out(agg2p, degp, hroot, bias2_2d):
    BN = 1000
    return pl.pallas_call(
        _out_body,
        grid=(N // BN,),
        in_specs=[
            pl.BlockSpec((NC, BN, H2), lambda i: (0, i, 0)),
            pl.BlockSpec((NC, BN, DW), lambda i: (0, i, 0)),
            pl.BlockSpec((BN, H2), lambda i: (i, 0)),
            pl.BlockSpec((1, H2), lambda i: (0, 0)),
        ],
        out_specs=pl.BlockSpec((BN, H2), lambda i: (i, 0)),
        out_shape=jax.ShapeDtypeStruct((N, H2), jnp.float32),
    )(agg2p, degp, hroot, bias2_2d)


def _sc_agg_body(width, conv1, gidx_hbm, dst_hbm, table_hbm, zrows_hbm,
                 zdeg_hbm, ones_hbm, agg_out, deg_out,
                 gidx_v, dst_v, rows_v, ones_v, acc_sh, deg_sh, sem0, sem1):
    c = lax.axis_index("c")
    s = lax.axis_index("s")
    w = c * NS + s
    sems = (sem0, sem1)

    pltpu.sync_copy(gidx_hbm.at[w], gidx_v.at[pl.ds(0, NCHUNK)])
    pltpu.sync_copy(dst_hbm.at[w], dst_v)

    pltpu.sync_copy(zrows_hbm, acc_sh.at[pl.ds(s * RPT, RPT)])
    if conv1:
        pltpu.sync_copy(zdeg_hbm, deg_sh.at[pl.ds(s * RPT, RPT)])
        pltpu.sync_copy(ones_hbm, ones_v)

    zi = jnp.zeros((LANES,), jnp.int32)
    for j in range(CHUNK // LANES):
        gidx_v[NCHUNK, pl.ds(j * LANES, LANES)] = zi
        gidx_v[NCHUNK + 1, pl.ds(j * LANES, LANES)] = zi

    plsc.subcore_barrier()

    for b in range(2):
        pltpu.async_copy(table_hbm.at[gidx_v.at[b]], rows_v.at[b], sems[b])

    def main_body(i, carry):
        k0 = i * 2
        for b in range(2):
            k = k0 + b
            pltpu.make_async_copy(
                table_hbm.at[gidx_v.at[k]], rows_v.at[b], sems[b]).wait()
            pltpu.sync_copy(rows_v.at[b], acc_sh.at[dst_v.at[k]], add=True)
            if conv1:
                pltpu.sync_copy(ones_v, deg_sh.at[dst_v.at[k]], add=True)
            pltpu.async_copy(
                table_hbm.at[gidx_v.at[k + 2]], rows_v.at[b], sems[b])
        return carry

    lax.fori_loop(0, NCHUNK // 2, main_body, 0)

    for b in range(2):
        pltpu.make_async_copy(
            table_hbm.at[gidx_v.at[NCHUNK]], rows_v.at[b], sems[b]).wait()

    plsc.subcore_barrier()
    pltpu.sync_copy(acc_sh.at[pl.ds(s * RPT, RPT)],
                    agg_out.at[pl.ds(c * NP + s * RPT, RPT)])
    if conv1:
        pltpu.sync_copy(deg_sh.at[pl.ds(s * RPT, RPT)],
                        deg_out.at[pl.ds(c * NP + s * RPT, RPT)])


def _make_sc_agg(width, conv1):
    out_type = [jax.ShapeDtypeStruct((NC * NP, width), jnp.float32)]
    scratch = [
        pltpu.VMEM((NCHUNK + 2, CHUNK), jnp.int32),
        pltpu.VMEM((NCHUNK, CHUNK), jnp.int32),
        pltpu.VMEM((2, CHUNK, width), jnp.float32),
        pltpu.VMEM((CHUNK, DW), jnp.float32),
        pltpu.VMEM_SHARED((NP, width), jnp.float32),
    ]
    if conv1:
        out_type.append(jax.ShapeDtypeStruct((NC * NP, DW), jnp.float32))
        scratch.append(pltpu.VMEM_SHARED((NP, DW), jnp.float32))
        body = functools.partial(_sc_agg_body, width, True)
    else:
        scratch.append(None)

        def body(gidx_hbm, dst_hbm, table_hbm, zrows_hbm, agg_out,
                 gidx_v, dst_v, rows_v, ones_v, acc_sh, sem0, sem1):
            _sc_agg_body(width, False, gidx_hbm, dst_hbm, table_hbm,
                         zrows_hbm, None, None, agg_out, None,
                         gidx_v, dst_v, rows_v, ones_v, acc_sh, None,
                         sem0, sem1)

    scratch = [sc for sc in scratch if sc is not None]
    scratch += [pltpu.SemaphoreType.DMA, pltpu.SemaphoreType.DMA]
    return functools.partial(
        pl.kernel,
        out_type=out_type if conv1 else out_type[0],
        mesh=_mesh,
        compiler_params=pltpu.CompilerParams(use_tc_tiling_on_sc=False),
        scratch_types=scratch,
    )(body)


_sc_agg1 = _make_sc_agg(H1, True)
_sc_agg2 = _make_sc_agg(H2, False)


def kernel(x, edge_index, edge_types, edge_timestamps, basis1, comp1, root1,
           bias1, basis2, comp2, root2, bias2):
    del x, edge_timestamps

    src = edge_index[0]
    dst = edge_index[1]
    rel = edge_types

    zdeg = jnp.zeros((RPT, DW), jnp.float32)
    ones_rows = jnp.ones((CHUNK, DW), jnp.float32)

    w1hn = _build_w1(comp1, jnp.swapaxes(basis1, 1, 2))
    w1t = jnp.swapaxes(w1hn, 1, 2).reshape(R * N, H1)
    w2f = _build_w2(comp2, basis2.reshape(NB, H1 * H2))
    w2cat = w2f.reshape(R, H1, H2)

    g1r, dstr = _build_gidx(src.reshape(NW, EPW), rel.reshape(NW, EPW),
                            dst.reshape(NW, EPW))

    zrows1 = jnp.zeros((RPT, H1), jnp.float32)

    agg1p, degp = _sc_agg1(g1r, dstr, w1t, zrows1, zdeg, ones_rows)
    agg1p = agg1p.reshape(NC, NP, H1)
    degp = degp.reshape(NC, NP, DW)

    xw3, hroot = _build_h1(agg1p, degp, root1, bias1.reshape(1, H1), w2cat,
                           root2)
    xwt = xw3.reshape(R * N, H2)

    zrows2 = jnp.zeros((RPT, H2), jnp.float32)
    agg2p = _sc_agg2(g1r, dstr, xwt, zrows2)
    agg2p = agg2p.reshape(NC, NP, H2)

    return _build_out(agg2p, degp, hroot, bias2.reshape(1, H2))


# speedup vs baseline: 1.0870x; 1.0870x over previous
import functools

import jax
import jax.numpy as jnp
from jax import lax
from jax.experimental import pallas as pl
from jax.experimental.pallas import tpu as pltpu
from jax.experimental.pallas import tpu_sc as plsc

N = 10000
E = 640000
R = 8
NB = 30
H1 = 64
H2 = 32

NC = 2
NS = 16
NW = NC * NS
EPW = E // NW
CHUNK = 80
NCHUNK = EPW // CHUNK
NBUF = 4
NCHP = 252
NP = 10240
RPT = NP // NS
LANES = 16
DW = 8

_mesh = plsc.VectorSubcoreMesh(
    core_axis_name="c", subcore_axis_name="s", num_cores=NC, num_subcores=NS)


BH = 8


def _w1_body(comp1_ref, basis_ref, out_ref):
    for hh in range(BH):
        out_ref[:, hh, :] = jnp.dot(comp1_ref[...], basis_ref[:, hh, :],
                                    preferred_element_type=jnp.float32)


def _build_w1(comp1, basis1_t):
    return pl.pallas_call(
        _w1_body,
        grid=(H1 // BH,),
        in_specs=[
            pl.BlockSpec((R, NB), lambda j: (0, 0)),
            pl.BlockSpec((NB, BH, N), lambda j: (0, j, 0)),
        ],
        out_specs=pl.BlockSpec((R, BH, N), lambda j: (0, j, 0)),
        out_shape=jax.ShapeDtypeStruct((R, H1, N), jnp.float32),
    )(comp1, basis1_t)


def _w2_body(comp2_ref, basis_ref, out_ref):
    out_ref[...] = jnp.dot(comp2_ref[...], basis_ref[...],
                           preferred_element_type=jnp.float32)


def _build_w2(comp2, basis2_flat):
    return pl.pallas_call(
        _w2_body,
        out_shape=jax.ShapeDtypeStruct((R, H1 * H2), jnp.float32),
    )(comp2, basis2_flat)


def _gidx_body(src_ref, rel_ref, g1_ref):
    g1_ref[...] = rel_ref[...] * N + src_ref[...]


def _build_gidx(src2d, rel2d):
    ROWS = E // 128
    BN = 1000
    return pl.pallas_call(
        _gidx_body,
        grid=(ROWS // BN,),
        in_specs=[
            pl.BlockSpec((BN, 128), lambda i: (i, 0)),
            pl.BlockSpec((BN, 128), lambda i: (i, 0)),
        ],
        out_specs=pl.BlockSpec((BN, 128), lambda i: (i, 0)),
        out_shape=jax.ShapeDtypeStruct((ROWS, 128), jnp.int32),
    )(src2d, rel2d)


def _h1_body(aggp_ref, degp_ref, root1_ref, bias1_ref, w2cat_ref, root2_ref,
             xw_ref, hroot_ref):
    dcol = (degp_ref[0] + degp_ref[1])[:, 0:1]
    invd = 1.0 / jnp.maximum(dcol, 1.0)
    a = aggp_ref[0] + aggp_ref[1]
    h1 = jnp.maximum(a * invd + root1_ref[...] + bias1_ref[...], 0.0)
    for r in range(R):
        xw_ref[r] = jnp.dot(h1, w2cat_ref[r], preferred_element_type=jnp.float32)
    hroot_ref[...] = jnp.dot(h1, root2_ref[...], preferred_element_type=jnp.float32)


def _build_h1(aggp, degp, root1, bias1_2d, w2cat, root2):
    BN = 1000
    return pl.pallas_call(
        _h1_body,
        grid=(N // BN,),
        in_specs=[
            pl.BlockSpec((NC, BN, H1), lambda i: (0, i, 0)),
            pl.BlockSpec((NC, BN, DW), lambda i: (0, i, 0)),
            pl.BlockSpec((BN, H1), lambda i: (i, 0)),
            pl.BlockSpec((1, H1), lambda i: (0, 0)),
            pl.BlockSpec((R, H1, H2), lambda i: (0, 0, 0)),
            pl.BlockSpec((H1, H2), lambda i: (0, 0)),
        ],
        out_specs=[
            pl.BlockSpec((R, BN, H2), lambda i: (0, i, 0)),
            pl.BlockSpec((BN, H2), lambda i: (i, 0)),
        ],
        out_shape=[
            jax.ShapeDtypeStruct((R, N, H2), jnp.float32),
            jax.ShapeDtypeStruct((N, H2), jnp.float32),
        ],
    )(aggp, degp, root1, bias1_2d, w2cat, root2)


def _out_body(agg2p_ref, degp_ref, hroot_ref, bias2_ref, out_ref):
    dcol = (degp_ref[0] + degp_ref[1])[:, 0:1]
    invd = 1.0 / jnp.maximum(dcol, 1.0)
    out_ref[...] = ((agg2p_ref[0] + agg2p_ref[1]) * invd
                    + hroot_ref[...] + bias2_ref[...])


def _build_out(agg2p, degp, hroot, bias2_2d):
    BN = 1000
    return pl.pallas_call(
        _out_body,
        grid=(N // BN,),
        in_specs=[
            pl.BlockSpec((NC, BN, H2), lambda i: (0, i, 0)),
            pl.BlockSpec((NC, BN, DW), lambda i: (0, i, 0)),
            pl.BlockSpec((BN, H2), lambda i: (i, 0)),
            pl.BlockSpec((1, H2), lambda i: (0, 0)),
        ],
        out_specs=pl.BlockSpec((BN, H2), lambda i: (i, 0)),
        out_shape=jax.ShapeDtypeStruct((N, H2), jnp.float32),
    )(agg2p, degp, hroot, bias2_2d)


def _sc_agg_body(width, conv1, gidx_hbm, dst_hbm, table_hbm, zrows_hbm,
                 zdeg_hbm, ones_hbm, agg_out, deg_out,
                 gidx_v, dst_v, rows_v, ones_v, acc_sh, deg_sh, *sems):
    c = lax.axis_index("c")
    s = lax.axis_index("s")
    w = c * NS + s

    pltpu.sync_copy(gidx_hbm.at[w], gidx_v.at[pl.ds(0, NCHUNK)])
    pltpu.sync_copy(dst_hbm.at[w], dst_v.at[pl.ds(0, NCHUNK)])

    pltpu.sync_copy(zrows_hbm, acc_sh.at[pl.ds(s * RPT, RPT)])
    if conv1:
        pltpu.sync_copy(zdeg_hbm, deg_sh.at[pl.ds(s * RPT, RPT)])
        pltpu.sync_copy(ones_hbm, ones_v)

    zi = jnp.zeros((LANES,), jnp.int32)
    ti = jnp.full((LANES,), NP - 1, jnp.int32)
    for k in range(NCHUNK, NCHP + NBUF):
        for j in range(CHUNK // LANES):
            gidx_v[k, pl.ds(j * LANES, LANES)] = zi
    for k in range(NCHUNK, NCHP):
        for j in range(CHUNK // LANES):
            dst_v[k, pl.ds(j * LANES, LANES)] = ti

    plsc.subcore_barrier()

    for b in range(NBUF):
        pltpu.async_copy(table_hbm.at[gidx_v.at[b]], rows_v.at[b], sems[b])

    def main_body(i, carry):
        k0 = i * NBUF
        for b in range(NBUF):
            k = k0 + b
            pltpu.make_async_copy(
                table_hbm.at[gidx_v.at[k]], rows_v.at[b], sems[b]).wait()
            pltpu.sync_copy(rows_v.at[b], acc_sh.at[dst_v.at[k]], add=True)
            if conv1:
                pltpu.sync_copy(ones_v, deg_sh.at[dst_v.at[k]], add=True)
            pltpu.async_copy(
                table_hbm.at[gidx_v.at[k + NBUF]], rows_v.at[b], sems[b])
        return carry

    lax.fori_loop(0, NCHP // NBUF, main_body, 0)

    for b in range(NBUF):
        pltpu.make_async_copy(
            table_hbm.at[gidx_v.at[NCHUNK]], rows_v.at[b], sems[b]).wait()

    plsc.subcore_barrier()
    pltpu.sync_copy(acc_sh.at[pl.ds(s * RPT, RPT)],
                    agg_out.at[pl.ds(c * NP + s * RPT, RPT)])
    if conv1:
        pltpu.sync_copy(deg_sh.at[pl.ds(s * RPT, RPT)],
                        deg_out.at[pl.ds(c * NP + s * RPT, RPT)])


def _make_sc_agg(width, conv1):
    out_type = [jax.ShapeDtypeStruct((NC * NP, width), jnp.float32)]
    scratch = [
        pltpu.VMEM((NCHP + NBUF, CHUNK), jnp.int32),
        pltpu.VMEM((NCHP, CHUNK), jnp.int32),
        pltpu.VMEM((NBUF, CHUNK, width), jnp.float32),
        pltpu.VMEM((CHUNK, DW), jnp.float32),
        pltpu.VMEM_SHARED((NP, width), jnp.float32),
    ]
    if conv1:
        out_type.append(jax.ShapeDtypeStruct((NC * NP, DW), jnp.float32))
        scratch.append(pltpu.VMEM_SHARED((NP, DW), jnp.float32))
        body = functools.partial(_sc_agg_body, width, True)
    else:
        scratch.append(None)

        def body(gidx_hbm, dst_hbm, table_hbm, zrows_hbm, agg_out,
                 gidx_v, dst_v, rows_v, ones_v, acc_sh, *sems):
            _sc_agg_body(width, False, gidx_hbm, dst_hbm, table_hbm,
                         zrows_hbm, None, None, agg_out, None,
                         gidx_v, dst_v, rows_v, ones_v, acc_sh, None,
                         *sems)

    scratch = [sc for sc in scratch if sc is not None]
    scratch += [pltpu.SemaphoreType.DMA] * NBUF
    return functools.partial(
        pl.kernel,
        out_type=out_type if conv1 else out_type[0],
        mesh=_mesh,
        compiler_params=pltpu.CompilerParams(use_tc_tiling_on_sc=False),
        scratch_types=scratch,
    )(body)


_sc_agg1 = _make_sc_agg(H1, True)
_sc_agg2 = _make_sc_agg(H2, False)


def kernel(x, edge_index, edge_types, edge_timestamps, basis1, comp1, root1,
           bias1, basis2, comp2, root2, bias2):
    del x, edge_timestamps

    src = edge_index[0]
    dst = edge_index[1]
    rel = edge_types

    zdeg = jnp.zeros((RPT, DW), jnp.float32)
    ones_rows = jnp.ones((CHUNK, DW), jnp.float32)

    w1hn = _build_w1(comp1, jnp.swapaxes(basis1, 1, 2))
    w1t = jnp.swapaxes(w1hn, 1, 2).reshape(R * N, H1)
    w2f = _build_w2(comp2, basis2.reshape(NB, H1 * H2))
    w2cat = w2f.reshape(R, H1, H2)

    g1 = _build_gidx(src.reshape(E // 128, 128), rel.reshape(E // 128, 128))
    g1r = g1.reshape(NW, NCHUNK, CHUNK)
    dstr = dst.reshape(NW, NCHUNK, CHUNK)

    zrows1 = jnp.zeros((RPT, H1), jnp.float32)

    agg1p, degp = _sc_agg1(g1r, dstr, w1t, zrows1, zdeg, ones_rows)
    agg1p = agg1p.reshape(NC, NP, H1)
    degp = degp.reshape(NC, NP, DW)

    xw3, hroot = _build_h1(agg1p, degp, root1, bias1.reshape(1, H1), w2cat,
                           root2)
    xwt = xw3.reshape(R * N, H2)

    zrows2 = jnp.zeros((RPT, H2), jnp.float32)
    agg2p = _sc_agg2(g1r, dstr, xwt, zrows2)
    agg2p = agg2p.reshape(NC, NP, H2)

    return _build_out(agg2p, degp, hroot, bias2.reshape(1, H2))


# speedup vs baseline: 1.2048x; 1.1084x over previous
import functools

import jax
import jax.numpy as jnp
from jax import lax
from jax.experimental import pallas as pl
from jax.experimental.pallas import tpu as pltpu
from jax.experimental.pallas import tpu_sc as plsc

N = 10000
E = 640000
R = 8
NB = 30
H1 = 64
H2 = 32

NC = 2
NS = 16
NW = NC * NS
EPW = E // NW
CHUNK = 80
NCHUNK = EPW // CHUNK
NBUF = 4
NCHP = 252
NP = 10240
RPT = NP // NS
LANES = 16
DW = 8

_mesh = plsc.VectorSubcoreMesh(
    core_axis_name="c", subcore_axis_name="s", num_cores=NC, num_subcores=NS)


BH = 8


def _w1_body(comp1_ref, basis_ref, out_ref):
    for hh in range(BH):
        out_ref[:, hh, :] = jnp.dot(comp1_ref[...], basis_ref[:, hh, :],
                                    preferred_element_type=jnp.float32)


def _build_w1(comp1, basis1_t):
    return pl.pallas_call(
        _w1_body,
        grid=(H1 // BH,),
        in_specs=[
            pl.BlockSpec((R, NB), lambda j: (0, 0)),
            pl.BlockSpec((NB, BH, N), lambda j: (0, j, 0)),
        ],
        out_specs=pl.BlockSpec((R, BH, N), lambda j: (0, j, 0)),
        out_shape=jax.ShapeDtypeStruct((R, H1, N), jnp.float32),
    )(comp1, basis1_t)


def _w2_body(comp2_ref, basis_ref, out_ref):
    out_ref[...] = jnp.dot(comp2_ref[...], basis_ref[...],
                           preferred_element_type=jnp.float32)


def _build_w2(comp2, basis2_flat):
    return pl.pallas_call(
        _w2_body,
        out_shape=jax.ShapeDtypeStruct((R, H1 * H2), jnp.float32),
    )(comp2, basis2_flat)


def _gidx_body(src_ref, rel_ref, g1_ref):
    g1_ref[...] = rel_ref[...] * N + src_ref[...]


def _build_gidx(src2d, rel2d):
    ROWS = E // 128
    BN = 1000
    return pl.pallas_call(
        _gidx_body,
        grid=(ROWS // BN,),
        in_specs=[
            pl.BlockSpec((BN, 128), lambda i: (i, 0)),
            pl.BlockSpec((BN, 128), lambda i: (i, 0)),
        ],
        out_specs=pl.BlockSpec((BN, 128), lambda i: (i, 0)),
        out_shape=jax.ShapeDtypeStruct((ROWS, 128), jnp.int32),
    )(src2d, rel2d)


def _h1_body(aggp_ref, degp_ref, root1_ref, bias1_ref, w2cat_ref, root2_ref,
             xw_ref, hroot_ref):
    dcol = (degp_ref[0] + degp_ref[1])[:, 0:1]
    invd = 1.0 / jnp.maximum(dcol, 1.0)
    a = aggp_ref[0] + aggp_ref[1]
    h1 = jnp.maximum(a * invd + root1_ref[...] + bias1_ref[...], 0.0)
    for r in range(R):
        xw_ref[r] = jnp.dot(h1, w2cat_ref[r], preferred_element_type=jnp.float32)
    hroot_ref[...] = jnp.dot(h1, root2_ref[...], preferred_element_type=jnp.float32)


def _build_h1(aggp, degp, root1, bias1_2d, w2cat, root2):
    BN = 1000
    return pl.pallas_call(
        _h1_body,
        grid=(N // BN,),
        in_specs=[
            pl.BlockSpec((NC, BN, H1), lambda i: (0, i, 0)),
            pl.BlockSpec((NC, BN, DW), lambda i: (0, i, 0)),
            pl.BlockSpec((BN, H1), lambda i: (i, 0)),
            pl.BlockSpec((1, H1), lambda i: (0, 0)),
            pl.BlockSpec((R, H1, H2), lambda i: (0, 0, 0)),
            pl.BlockSpec((H1, H2), lambda i: (0, 0)),
        ],
        out_specs=[
            pl.BlockSpec((R, BN, H2), lambda i: (0, i, 0)),
            pl.BlockSpec((BN, H2), lambda i: (i, 0)),
        ],
        out_shape=[
            jax.ShapeDtypeStruct((R, N, H2), jnp.float32),
            jax.ShapeDtypeStruct((N, H2), jnp.float32),
        ],
    )(aggp, degp, root1, bias1_2d, w2cat, root2)


def _out_body(agg2p_ref, degp_ref, hroot_ref, bias2_ref, out_ref):
    dcol = (degp_ref[0] + degp_ref[1])[:, 0:1]
    invd = 1.0 / jnp.maximum(dcol, 1.0)
    out_ref[...] = ((agg2p_ref[0] + agg2p_ref[1]) * invd
                    + hroot_ref[...] + bias2_ref[...])


def _build_out(agg2p, degp, hroot, bias2_2d):
    BN = 1000
    return pl.pallas_call(
        _out_body,
        grid=(N // BN,),
        in_specs=[
            pl.BlockSpec((NC, BN, H2), lambda i: (0, i, 0)),
            pl.BlockSpec((NC, BN, DW), lambda i: (0, i, 0)),
            pl.BlockSpec((BN, H2), lambda i: (i, 0)),
            pl.BlockSpec((1, H2), lambda i: (0, 0)),
        ],
        out_specs=pl.BlockSpec((BN, H2), lambda i: (i, 0)),
        out_shape=jax.ShapeDtypeStruct((N, H2), jnp.float32),
    )(agg2p, degp, hroot, bias2_2d)


def _sc_agg_body(width, conv1, gidx_hbm, dst_hbm, table_hbm, zrows_hbm,
                 zdeg_hbm, ones_hbm, agg_out, deg_out,
                 gidx_v, dst_v, rows_v, ones_v, acc_sh, deg_sh, *sems):
    c = lax.axis_index("c")
    s = lax.axis_index("s")
    w = c * NS + s

    pltpu.sync_copy(gidx_hbm.at[w], gidx_v.at[pl.ds(0, NCHUNK)])
    pltpu.sync_copy(dst_hbm.at[w], dst_v.at[pl.ds(0, NCHUNK)])

    pltpu.sync_copy(zrows_hbm, acc_sh.at[pl.ds(s * RPT, RPT)])
    if conv1:
        pltpu.sync_copy(zdeg_hbm, deg_sh.at[pl.ds(s * RPT, RPT)])
        pltpu.sync_copy(ones_hbm, ones_v)

    zi = jnp.zeros((LANES,), jnp.int32)
    ti = jnp.full((LANES,), NP - 1, jnp.int32)
    for k in range(NCHUNK, NCHP + NBUF):
        for j in range(CHUNK // LANES):
            gidx_v[k, pl.ds(j * LANES, LANES)] = zi
    for k in range(NCHUNK, NCHP):
        for j in range(CHUNK // LANES):
            dst_v[k, pl.ds(j * LANES, LANES)] = ti

    plsc.subcore_barrier()

    sem_g = sems[0:NBUF]
    sem_s = sems[NBUF:2 * NBUF]
    sem_o = sems[2 * NBUF:3 * NBUF] if conv1 else None

    pltpu.async_copy(table_hbm.at[gidx_v.at[0]], rows_v.at[0], sem_g[0])
    pltpu.async_copy(table_hbm.at[gidx_v.at[1]], rows_v.at[1], sem_g[1])
    for b in (2, 3):
        pltpu.async_copy(rows_v.at[b], acc_sh.at[dst_v.at[NCHUNK]],
                         sem_s[b], add=True)
        if conv1:
            pltpu.async_copy(ones_v, deg_sh.at[dst_v.at[NCHUNK]],
                             sem_o[b], add=True)

    def main_body(i, carry):
        k0 = i * NBUF
        for b in range(NBUF):
            k = k0 + b
            bq = (b + 2) % NBUF
            pltpu.make_async_copy(
                table_hbm.at[gidx_v.at[k]], rows_v.at[b], sem_g[b]).wait()
            pltpu.async_copy(rows_v.at[b], acc_sh.at[dst_v.at[k]],
                             sem_s[b], add=True)
            if conv1:
                pltpu.async_copy(ones_v, deg_sh.at[dst_v.at[k]],
                                 sem_o[b], add=True)
            pltpu.make_async_copy(rows_v.at[bq], acc_sh.at[dst_v.at[k]],
                                  sem_s[bq]).wait()
            if conv1:
                pltpu.make_async_copy(ones_v, deg_sh.at[dst_v.at[k]],
                                      sem_o[bq]).wait()
            pltpu.async_copy(
                table_hbm.at[gidx_v.at[k + 2]], rows_v.at[bq], sem_g[bq])
        return carry

    lax.fori_loop(0, NCHP // NBUF, main_body, 0)

    for b in (0, 1):
        pltpu.make_async_copy(
            table_hbm.at[gidx_v.at[NCHUNK]], rows_v.at[b], sem_g[b]).wait()
    for b in (2, 3):
        pltpu.make_async_copy(rows_v.at[b], acc_sh.at[dst_v.at[NCHUNK]],
                              sem_s[b]).wait()
        if conv1:
            pltpu.make_async_copy(ones_v, deg_sh.at[dst_v.at[NCHUNK]],
                                  sem_o[b]).wait()

    plsc.subcore_barrier()
    pltpu.sync_copy(acc_sh.at[pl.ds(s * RPT, RPT)],
                    agg_out.at[pl.ds(c * NP + s * RPT, RPT)])
    if conv1:
        pltpu.sync_copy(deg_sh.at[pl.ds(s * RPT, RPT)],
                        deg_out.at[pl.ds(c * NP + s * RPT, RPT)])


def _make_sc_agg(width, conv1):
    out_type = [jax.ShapeDtypeStruct((NC * NP, width), jnp.float32)]
    scratch = [
        pltpu.VMEM((NCHP + NBUF, CHUNK), jnp.int32),
        pltpu.VMEM((NCHP, CHUNK), jnp.int32),
        pltpu.VMEM((NBUF, CHUNK, width), jnp.float32),
        pltpu.VMEM((CHUNK, DW), jnp.float32),
        pltpu.VMEM_SHARED((NP, width), jnp.float32),
    ]
    if conv1:
        out_type.append(jax.ShapeDtypeStruct((NC * NP, DW), jnp.float32))
        scratch.append(pltpu.VMEM_SHARED((NP, DW), jnp.float32))
        body = functools.partial(_sc_agg_body, width, True)
    else:
        scratch.append(None)

        def body(gidx_hbm, dst_hbm, table_hbm, zrows_hbm, agg_out,
                 gidx_v, dst_v, rows_v, ones_v, acc_sh, *sems):
            _sc_agg_body(width, False, gidx_hbm, dst_hbm, table_hbm,
                         zrows_hbm, None, None, agg_out, None,
                         gidx_v, dst_v, rows_v, ones_v, acc_sh, None,
                         *sems)

    scratch = [sc for sc in scratch if sc is not None]
    scratch += [pltpu.SemaphoreType.DMA] * (NBUF * (3 if conv1 else 2))
    return functools.partial(
        pl.kernel,
        out_type=out_type if conv1 else out_type[0],
        mesh=_mesh,
        compiler_params=pltpu.CompilerParams(use_tc_tiling_on_sc=False),
        scratch_types=scratch,
    )(body)


_sc_agg1 = _make_sc_agg(H1, True)
_sc_agg2 = _make_sc_agg(H2, False)


def kernel(x, edge_index, edge_types, edge_timestamps, basis1, comp1, root1,
           bias1, basis2, comp2, root2, bias2):
    del x, edge_timestamps

    src = edge_index[0]
    dst = edge_index[1]
    rel = edge_types

    zdeg = jnp.zeros((RPT, DW), jnp.float32)
    ones_rows = jnp.ones((CHUNK, DW), jnp.float32)

    w1hn = _build_w1(comp1, jnp.swapaxes(basis1, 1, 2))
    w1t = jnp.swapaxes(w1hn, 1, 2).reshape(R * N, H1)
    w2f = _build_w2(comp2, basis2.reshape(NB, H1 * H2))
    w2cat = w2f.reshape(R, H1, H2)

    g1 = _build_gidx(src.reshape(E // 128, 128), rel.reshape(E // 128, 128))
    g1r = g1.reshape(NW, NCHUNK, CHUNK)
    dstr = dst.reshape(NW, NCHUNK, CHUNK)

    zrows1 = jnp.zeros((RPT, H1), jnp.float32)

    agg1p, degp = _sc_agg1(g1r, dstr, w1t, zrows1, zdeg, ones_rows)
    agg1p = agg1p.reshape(NC, NP, H1)
    degp = degp.reshape(NC, NP, DW)

    xw3, hroot = _build_h1(agg1p, degp, root1, bias1.reshape(1, H1), w2cat,
                           root2)
    xwt = xw3.reshape(R * N, H2)

    zrows2 = jnp.zeros((RPT, H2), jnp.float32)
    agg2p = _sc_agg2(g1r, dstr, xwt, zrows2)
    agg2p = agg2p.reshape(NC, NP, H2)

    return _build_out(agg2p, degp, hroot, bias2.reshape(1, H2))


# speedup vs baseline: 1.4315x; 1.1882x over previous
import functools

import jax
import jax.numpy as jnp
from jax import lax
from jax.experimental import pallas as pl
from jax.experimental.pallas import tpu as pltpu
from jax.experimental.pallas import tpu_sc as plsc

N = 10000
E = 640000
R = 8
NB = 30
H1 = 64
H2 = 32

NC = 2
NS = 16
NW = NC * NS
EPW = E // NW
CHUNK = 80
NCHUNK = EPW // CHUNK
NBUF = 2
NP = 10240
RPT = NP // NS
LANES = 16
DW = 8

_mesh = plsc.VectorSubcoreMesh(
    core_axis_name="c", subcore_axis_name="s", num_cores=NC, num_subcores=NS)


BH = 8


def _w1_body(comp1_ref, basis_ref, out_ref):
    for hh in range(BH):
        out_ref[:, hh, :] = jnp.dot(comp1_ref[...], basis_ref[:, hh, :],
                                    preferred_element_type=jnp.float32)


def _build_w1(comp1, basis1_t):
    return pl.pallas_call(
        _w1_body,
        grid=(H1 // BH,),
        in_specs=[
            pl.BlockSpec((R, NB), lambda j: (0, 0)),
            pl.BlockSpec((NB, BH, N), lambda j: (0, j, 0)),
        ],
        out_specs=pl.BlockSpec((R, BH, N), lambda j: (0, j, 0)),
        out_shape=jax.ShapeDtypeStruct((R, H1, N), jnp.float32),
    )(comp1, basis1_t)


def _w2_body(comp2_ref, basis_ref, out_ref):
    out_ref[...] = jnp.dot(comp2_ref[...], basis_ref[...],
                           preferred_element_type=jnp.float32)


def _build_w2(comp2, basis2_flat):
    return pl.pallas_call(
        _w2_body,
        out_shape=jax.ShapeDtypeStruct((R, H1 * H2), jnp.float32),
    )(comp2, basis2_flat)


def _gidx_body(src_ref, rel_ref, g1_ref):
    g1_ref[...] = rel_ref[...] * N + src_ref[...]


def _build_gidx(src2d, rel2d):
    ROWS = E // 128
    BN = 1000
    return pl.pallas_call(
        _gidx_body,
        grid=(ROWS // BN,),
        in_specs=[
            pl.BlockSpec((BN, 128), lambda i: (i, 0)),
            pl.BlockSpec((BN, 128), lambda i: (i, 0)),
        ],
        out_specs=pl.BlockSpec((BN, 128), lambda i: (i, 0)),
        out_shape=jax.ShapeDtypeStruct((ROWS, 128), jnp.int32),
    )(src2d, rel2d)


def _h1_body(aggp_ref, degp_ref, root1_ref, bias1_ref, w2cat_ref, root2_ref,
             xw_ref, hroot_ref):
    dcol = (degp_ref[0] + degp_ref[1])[:, 0:1]
    invd = 1.0 / jnp.maximum(dcol, 1.0)
    a = aggp_ref[0] + aggp_ref[1]
    h1 = jnp.maximum(a * invd + root1_ref[...] + bias1_ref[...], 0.0)
    for r in range(R):
        xw_ref[r] = jnp.dot(h1, w2cat_ref[r], preferred_element_type=jnp.float32)
    hroot_ref[...] = jnp.dot(h1, root2_ref[...], preferred_element_type=jnp.float32)


def _build_h1(aggp, degp, root1, bias1_2d, w2cat, root2):
    BN = 1000
    return pl.pallas_call(
        _h1_body,
        grid=(N // BN,),
        in_specs=[
            pl.BlockSpec((NC, BN, H1), lambda i: (0, i, 0)),
            pl.BlockSpec((NC, BN, DW), lambda i: (0, i, 0)),
            pl.BlockSpec((BN, H1), lambda i: (i, 0)),
            pl.BlockSpec((1, H1), lambda i: (0, 0)),
            pl.BlockSpec((R, H1, H2), lambda i: (0, 0, 0)),
            pl.BlockSpec((H1, H2), lambda i: (0, 0)),
        ],
        out_specs=[
            pl.BlockSpec((R, BN, H2), lambda i: (0, i, 0)),
            pl.BlockSpec((BN, H2), lambda i: (i, 0)),
        ],
        out_shape=[
            jax.ShapeDtypeStruct((R, N, H2), jnp.float32),
            jax.ShapeDtypeStruct((N, H2), jnp.float32),
        ],
    )(aggp, degp, root1, bias1_2d, w2cat, root2)


def _out_body(agg2p_ref, degp_ref, hroot_ref, bias2_ref, out_ref):
    dcol = (degp_ref[0] + degp_ref[1])[:, 0:1]
    invd = 1.0 / jnp.maximum(dcol, 1.0)
    out_ref[...] = ((agg2p_ref[0] + agg2p_ref[1]) * invd
                    + hroot_ref[...] + bias2_ref[...])


def _build_out(agg2p, degp, hroot, bias2_2d):
    BN = 1000
    return pl.pallas_call(
        _out_body,
        grid=(N // BN,),
        in_specs=[
            pl.BlockSpec((NC, BN, H2), lambda i: (0, i, 0)),
            pl.BlockSpec((NC, BN, DW), lambda i: (0, i, 0)),
            pl.BlockSpec((BN, H2), lambda i: (i, 0)),
            pl.BlockSpec((1, H2), lambda i: (0, 0)),
        ],
        out_specs=pl.BlockSpec((BN, H2), lambda i: (i, 0)),
        out_shape=jax.ShapeDtypeStruct((N, H2), jnp.float32),
    )(agg2p, degp, hroot, bias2_2d)


def _sc_agg_body(width, conv1, gidx_hbm, dst_hbm, table_hbm, zrows_hbm,
                 zdeg_hbm, ones_hbm, agg_out, deg_out,
                 gidx_v, dst_v, rows_v, ones_v, acc_sh, deg_sh, *sems):
    c = lax.axis_index("c")
    s = lax.axis_index("s")
    w = c * NS + s

    pltpu.sync_copy(gidx_hbm.at[w], gidx_v.at[pl.ds(0, NCHUNK)])
    pltpu.sync_copy(dst_hbm.at[w], dst_v.at[pl.ds(0, NCHUNK)])

    pltpu.sync_copy(zrows_hbm, acc_sh.at[pl.ds(s * RPT, RPT)])
    if conv1:
        pltpu.sync_copy(zdeg_hbm, deg_sh.at[pl.ds(s * RPT, RPT)])
        pltpu.sync_copy(ones_hbm, ones_v)

    zi = jnp.zeros((LANES,), jnp.int32)
    ti = jnp.full((LANES,), NP - 1, jnp.int32)
    for k in range(NCHUNK, NCHUNK + NBUF):
        for j in range(CHUNK // LANES):
            gidx_v[k, pl.ds(j * LANES, LANES)] = zi
    for j in range(CHUNK // LANES):
        dst_v[NCHUNK, pl.ds(j * LANES, LANES)] = ti

    plsc.subcore_barrier()

    sem_g = sems[0:NBUF]
    sem_o = sems[NBUF:2 * NBUF] if conv1 else None

    for b in range(NBUF):
        pltpu.async_copy(table_hbm.at[gidx_v.at[b]], rows_v.at[b], sem_g[b])
        if conv1:
            pltpu.async_copy(ones_v, deg_sh.at[dst_v.at[NCHUNK]],
                             sem_o[b], add=True)

    def main_body(i, carry):
        k0 = i * NBUF
        for b in range(NBUF):
            k = k0 + b
            pltpu.make_async_copy(
                table_hbm.at[gidx_v.at[k]], rows_v.at[b], sem_g[b]).wait()
            pltpu.sync_copy(rows_v.at[b], acc_sh.at[dst_v.at[k]], add=True)
            if conv1:
                pltpu.make_async_copy(ones_v, deg_sh.at[dst_v.at[k]],
                                      sem_o[b]).wait()
                pltpu.async_copy(ones_v, deg_sh.at[dst_v.at[k]],
                                 sem_o[b], add=True)
            pltpu.async_copy(
                table_hbm.at[gidx_v.at[k + NBUF]], rows_v.at[b], sem_g[b])
        return carry

    lax.fori_loop(0, NCHUNK // NBUF, main_body, 0)

    for b in range(NBUF):
        pltpu.make_async_copy(
            table_hbm.at[gidx_v.at[NCHUNK]], rows_v.at[b], sem_g[b]).wait()
        if conv1:
            pltpu.make_async_copy(ones_v, deg_sh.at[dst_v.at[NCHUNK]],
                                  sem_o[b]).wait()

    plsc.subcore_barrier()
    pltpu.sync_copy(acc_sh.at[pl.ds(s * RPT, RPT)],
                    agg_out.at[pl.ds(c * NP + s * RPT, RPT)])
    if conv1:
        pltpu.sync_copy(deg_sh.at[pl.ds(s * RPT, RPT)],
                        deg_out.at[pl.ds(c * NP + s * RPT, RPT)])


def _make_sc_agg(width, conv1):
    out_type = [jax.ShapeDtypeStruct((NC * NP, width), jnp.float32)]
    scratch = [
        pltpu.VMEM((NCHUNK + NBUF, CHUNK), jnp.int32),
        pltpu.VMEM((NCHUNK + 1, CHUNK), jnp.int32),
        pltpu.VMEM((NBUF, CHUNK, width), jnp.float32),
        pltpu.VMEM((CHUNK, DW), jnp.float32),
        pltpu.VMEM_SHARED((NP, width), jnp.float32),
    ]
    if conv1:
        out_type.append(jax.ShapeDtypeStruct((NC * NP, DW), jnp.float32))
        scratch.append(pltpu.VMEM_SHARED((NP, DW), jnp.float32))
        body = functools.partial(_sc_agg_body, width, True)
    else:
        scratch.append(None)

        def body(gidx_hbm, dst_hbm, table_hbm, zrows_hbm, agg_out,
                 gidx_v, dst_v, rows_v, ones_v, acc_sh, *sems):
            _sc_agg_body(width, False, gidx_hbm, dst_hbm, table_hbm,
                         zrows_hbm, None, None, agg_out, None,
                         gidx_v, dst_v, rows_v, ones_v, acc_sh, None,
                         *sems)

    scratch = [sc for sc in scratch if sc is not None]
    scratch += [pltpu.SemaphoreType.DMA] * (NBUF * (2 if conv1 else 1))
    return functools.partial(
        pl.kernel,
        out_type=out_type if conv1 else out_type[0],
        mesh=_mesh,
        compiler_params=pltpu.CompilerParams(use_tc_tiling_on_sc=False),
        scratch_types=scratch,
    )(body)


_sc_agg1 = _make_sc_agg(H1, True)
_sc_agg2 = _make_sc_agg(H2, False)


def kernel(x, edge_index, edge_types, edge_timestamps, basis1, comp1, root1,
           bias1, basis2, comp2, root2, bias2):
    del x, edge_timestamps

    src = edge_index[0]
    dst = edge_index[1]
    rel = edge_types

    zdeg = jnp.zeros((RPT, DW), jnp.float32)
    ones_rows = jnp.ones((CHUNK, DW), jnp.float32)

    w1hn = _build_w1(comp1, jnp.swapaxes(basis1, 1, 2))
    w1t = jnp.swapaxes(w1hn, 1, 2).reshape(R * N, H1)
    w2f = _build_w2(comp2, basis2.reshape(NB, H1 * H2))
    w2cat = w2f.reshape(R, H1, H2)

    g1 = _build_gidx(src.reshape(E // 128, 128), rel.reshape(E // 128, 128))
    g1r = g1.reshape(NW, NCHUNK, CHUNK)
    dstr = dst.reshape(NW, NCHUNK, CHUNK)

    zrows1 = jnp.zeros((RPT, H1), jnp.float32)

    agg1p, degp = _sc_agg1(g1r, dstr, w1t, zrows1, zdeg, ones_rows)
    agg1p = agg1p.reshape(NC, NP, H1)
    degp = degp.reshape(NC, NP, DW)

    xw3, hroot = _build_h1(agg1p, degp, root1, bias1.reshape(1, H1), w2cat,
                           root2)
    xwt = xw3.reshape(R * N, H2)

    zrows2 = jnp.zeros((RPT, H2), jnp.float32)
    agg2p = _sc_agg2(g1r, dstr, xwt, zrows2)
    agg2p = agg2p.reshape(NC, NP, H2)

    return _build_out(agg2p, degp, hroot, bias2.reshape(1, H2))


# speedup vs baseline: 1.4417x; 1.0071x over previous
import functools

import jax
import jax.numpy as jnp
from jax import lax
from jax.experimental import pallas as pl
from jax.experimental.pallas import tpu as pltpu
from jax.experimental.pallas import tpu_sc as plsc

N = 10000
E = 640000
R = 8
NB = 30
H1 = 64
H2 = 32

NC = 2
NS = 16
NW = NC * NS
EPW = E // NW
CHUNK = 80
NCHUNK = EPW // CHUNK
NBUF = 2
NP = 10240
RPT = NP // NS
LANES = 16
DW = 8

_mesh = plsc.VectorSubcoreMesh(
    core_axis_name="c", subcore_axis_name="s", num_cores=NC, num_subcores=NS)


BH = 8


def _w1_body(comp1_ref, basis_ref, out_ref):
    for hh in range(BH):
        out_ref[:, hh, :] = jnp.dot(comp1_ref[...], basis_ref[:, hh, :],
                                    preferred_element_type=jnp.float32)


def _build_w1(comp1, basis1_t):
    return pl.pallas_call(
        _w1_body,
        grid=(H1 // BH,),
        in_specs=[
            pl.BlockSpec((R, NB), lambda j: (0, 0)),
            pl.BlockSpec((NB, BH, N), lambda j: (0, j, 0)),
        ],
        out_specs=pl.BlockSpec((R, BH, N), lambda j: (0, j, 0)),
        out_shape=jax.ShapeDtypeStruct((R, H1, N), jnp.float32),
    )(comp1, basis1_t)


def _w2_body(comp2_ref, basis_ref, out_ref):
    out_ref[...] = jnp.dot(comp2_ref[...], basis_ref[...],
                           preferred_element_type=jnp.float32)


def _build_w2(comp2, basis2_flat):
    return pl.pallas_call(
        _w2_body,
        out_shape=jax.ShapeDtypeStruct((R, H1 * H2), jnp.float32),
    )(comp2, basis2_flat)


def _gidx_body(src_ref, rel_ref, g1_ref):
    g1_ref[...] = rel_ref[...] * N + src_ref[...]


def _build_gidx(src2d, rel2d):
    ROWS = E // 128
    BN = 1000
    return pl.pallas_call(
        _gidx_body,
        grid=(ROWS // BN,),
        in_specs=[
            pl.BlockSpec((BN, 128), lambda i: (i, 0)),
            pl.BlockSpec((BN, 128), lambda i: (i, 0)),
        ],
        out_specs=pl.BlockSpec((BN, 128), lambda i: (i, 0)),
        out_shape=jax.ShapeDtypeStruct((ROWS, 128), jnp.int32),
    )(src2d, rel2d)


def _h1_body(aggp_ref, degp_ref, root1_ref, bias1_ref, w2cat_ref, root2_ref,
             xw_ref, hroot_ref):
    dcol = (degp_ref[0] + degp_ref[1])[:, 0:1]
    invd = 1.0 / jnp.maximum(dcol, 1.0)
    a = aggp_ref[0] + aggp_ref[1]
    h1 = jnp.maximum(a * invd + root1_ref[...] + bias1_ref[...], 0.0)
    for r in range(R):
        xw_ref[r] = jnp.dot(h1, w2cat_ref[r], preferred_element_type=jnp.float32)
    hroot_ref[...] = jnp.dot(h1, root2_ref[...], preferred_element_type=jnp.float32)


def _build_h1(aggp, degp, root1, bias1_2d, w2cat, root2):
    BN = 2000
    return pl.pallas_call(
        _h1_body,
        grid=(N // BN,),
        in_specs=[
            pl.BlockSpec((NC, BN, H1), lambda i: (0, i, 0)),
            pl.BlockSpec((NC, BN, DW), lambda i: (0, i, 0)),
            pl.BlockSpec((BN, H1), lambda i: (i, 0)),
            pl.BlockSpec((1, H1), lambda i: (0, 0)),
            pl.BlockSpec((R, H1, H2), lambda i: (0, 0, 0)),
            pl.BlockSpec((H1, H2), lambda i: (0, 0)),
        ],
        out_specs=[
            pl.BlockSpec((R, BN, H2), lambda i: (0, i, 0)),
            pl.BlockSpec((BN, H2), lambda i: (i, 0)),
        ],
        out_shape=[
            jax.ShapeDtypeStruct((R, N, H2), jnp.float32),
            jax.ShapeDtypeStruct((N, H2), jnp.float32),
        ],
    )(aggp, degp, root1, bias1_2d, w2cat, root2)


def _out_body(agg2p_ref, degp_ref, hroot_ref, bias2_ref, out_ref):
    dcol = (degp_ref[0] + degp_ref[1])[:, 0:1]
    invd = 1.0 / jnp.maximum(dcol, 1.0)
    out_ref[...] = ((agg2p_ref[0] + agg2p_ref[1]) * invd
                    + hroot_ref[...] + bias2_ref[...])


def _build_out(agg2p, degp, hroot, bias2_2d):
    BN = 2000
    return pl.pallas_call(
        _out_body,
        grid=(N // BN,),
        in_specs=[
            pl.BlockSpec((NC, BN, H2), lambda i: (0, i, 0)),
            pl.BlockSpec((NC, BN, DW), lambda i: (0, i, 0)),
            pl.BlockSpec((BN, H2), lambda i: (i, 0)),
            pl.BlockSpec((1, H2), lambda i: (0, 0)),
        ],
        out_specs=pl.BlockSpec((BN, H2), lambda i: (i, 0)),
        out_shape=jax.ShapeDtypeStruct((N, H2), jnp.float32),
    )(agg2p, degp, hroot, bias2_2d)


def _sc_agg_body(width, conv1, gidx_hbm, dst_hbm, table_hbm, zrows_hbm,
                 zdeg_hbm, ones_hbm, agg_out, deg_out,
                 gidx_v, dst_v, rows_v, ones_v, acc_sh, deg_sh, *sems):
    c = lax.axis_index("c")
    s = lax.axis_index("s")
    w = c * NS + s

    pltpu.sync_copy(gidx_hbm.at[w], gidx_v.at[pl.ds(0, NCHUNK)])
    pltpu.sync_copy(dst_hbm.at[w], dst_v.at[pl.ds(0, NCHUNK)])

    pltpu.sync_copy(zrows_hbm, acc_sh.at[pl.ds(s * RPT, RPT)])
    if conv1:
        pltpu.sync_copy(zdeg_hbm, deg_sh.at[pl.ds(s * RPT, RPT)])
        pltpu.sync_copy(ones_hbm, ones_v)

    zi = jnp.zeros((LANES,), jnp.int32)
    ti = jnp.full((LANES,), NP - 1, jnp.int32)
    for k in range(NCHUNK, NCHUNK + NBUF):
        for j in range(CHUNK // LANES):
            gidx_v[k, pl.ds(j * LANES, LANES)] = zi
    for j in range(CHUNK // LANES):
        dst_v[NCHUNK, pl.ds(j * LANES, LANES)] = ti

    plsc.subcore_barrier()

    sem_g = sems[0:NBUF]
    sem_o = sems[NBUF:2 * NBUF] if conv1 else None

    for b in range(NBUF):
        pltpu.async_copy(table_hbm.at[gidx_v.at[b]], rows_v.at[b], sem_g[b])
        if conv1:
            pltpu.async_copy(ones_v, deg_sh.at[dst_v.at[NCHUNK]],
                             sem_o[b], add=True)

    def main_body(i, carry):
        k0 = i * NBUF
        for b in range(NBUF):
            k = k0 + b
            pltpu.make_async_copy(
                table_hbm.at[gidx_v.at[k]], rows_v.at[b], sem_g[b]).wait()
            pltpu.sync_copy(rows_v.at[b], acc_sh.at[dst_v.at[k]], add=True)
            if conv1:
                pltpu.make_async_copy(ones_v, deg_sh.at[dst_v.at[k]],
                                      sem_o[b]).wait()
                pltpu.async_copy(ones_v, deg_sh.at[dst_v.at[k]],
                                 sem_o[b], add=True)
            pltpu.async_copy(
                table_hbm.at[gidx_v.at[k + NBUF]], rows_v.at[b], sem_g[b])
        return carry

    lax.fori_loop(0, NCHUNK // NBUF, main_body, 0)

    for b in range(NBUF):
        pltpu.make_async_copy(
            table_hbm.at[gidx_v.at[NCHUNK]], rows_v.at[b], sem_g[b]).wait()
        if conv1:
            pltpu.make_async_copy(ones_v, deg_sh.at[dst_v.at[NCHUNK]],
                                  sem_o[b]).wait()

    plsc.subcore_barrier()
    pltpu.sync_copy(acc_sh.at[pl.ds(s * RPT, RPT)],
                    agg_out.at[pl.ds(c * NP + s * RPT, RPT)])
    if conv1:
        pltpu.sync_copy(deg_sh.at[pl.ds(s * RPT, RPT)],
                        deg_out.at[pl.ds(c * NP + s * RPT, RPT)])


def _make_sc_agg(width, conv1):
    out_type = [jax.ShapeDtypeStruct((NC * NP, width), jnp.float32)]
    scratch = [
        pltpu.VMEM((NCHUNK + NBUF, CHUNK), jnp.int32),
        pltpu.VMEM((NCHUNK + 1, CHUNK), jnp.int32),
        pltpu.VMEM((NBUF, CHUNK, width), jnp.float32),
        pltpu.VMEM((CHUNK, DW), jnp.float32),
        pltpu.VMEM_SHARED((NP, width), jnp.float32),
    ]
    if conv1:
        out_type.append(jax.ShapeDtypeStruct((NC * NP, DW), jnp.float32))
        scratch.append(pltpu.VMEM_SHARED((NP, DW), jnp.float32))
        body = functools.partial(_sc_agg_body, width, True)
    else:
        scratch.append(None)

        def body(gidx_hbm, dst_hbm, table_hbm, zrows_hbm, agg_out,
                 gidx_v, dst_v, rows_v, ones_v, acc_sh, *sems):
            _sc_agg_body(width, False, gidx_hbm, dst_hbm, table_hbm,
                         zrows_hbm, None, None, agg_out, None,
                         gidx_v, dst_v, rows_v, ones_v, acc_sh, None,
                         *sems)

    scratch = [sc for sc in scratch if sc is not None]
    scratch += [pltpu.SemaphoreType.DMA] * (NBUF * (2 if conv1 else 1))
    return functools.partial(
        pl.kernel,
        out_type=out_type if conv1 else out_type[0],
        mesh=_mesh,
        compiler_params=pltpu.CompilerParams(use_tc_tiling_on_sc=False),
        scratch_types=scratch,
    )(body)


_sc_agg1 = _make_sc_agg(H1, True)
_sc_agg2 = _make_sc_agg(H2, False)


def kernel(x, edge_index, edge_types, edge_timestamps, basis1, comp1, root1,
           bias1, basis2, comp2, root2, bias2):
    del x, edge_timestamps

    src = edge_index[0]
    dst = edge_index[1]
    rel = edge_types

    zdeg = jnp.zeros((RPT, DW), jnp.float32)
    ones_rows = jnp.ones((CHUNK, DW), jnp.float32)

    w1hn = _build_w1(comp1, jnp.swapaxes(basis1, 1, 2))
    w1t = jnp.swapaxes(w1hn, 1, 2).reshape(R * N, H1)
    w2f = _build_w2(comp2, basis2.reshape(NB, H1 * H2))
    w2cat = w2f.reshape(R, H1, H2)

    g1 = _build_gidx(src.reshape(E // 128, 128), rel.reshape(E // 128, 128))
    g1r = g1.reshape(NW, NCHUNK, CHUNK)
    dstr = dst.reshape(NW, NCHUNK, CHUNK)

    zrows1 = jnp.zeros((RPT, H1), jnp.float32)

    agg1p, degp = _sc_agg1(g1r, dstr, w1t, zrows1, zdeg, ones_rows)
    agg1p = agg1p.reshape(NC, NP, H1)
    degp = degp.reshape(NC, NP, DW)

    xw3, hroot = _build_h1(agg1p, degp, root1, bias1.reshape(1, H1), w2cat,
                           root2)
    xwt = xw3.reshape(R * N, H2)

    zrows2 = jnp.zeros((RPT, H2), jnp.float32)
    agg2p = _sc_agg2(g1r, dstr, xwt, zrows2)
    agg2p = agg2p.reshape(NC, NP, H2)

    return _build_out(agg2p, degp, hroot, bias2.reshape(1, H2))
